# baseline (speedup 1.0000x reference)
import jax
import jax.numpy as jnp
from jax import lax
from jax.experimental import pallas as pl
from jax.experimental.pallas import tpu as pltpu

_EPS = 1e-5


def _dims(H, W):
    Hp, Wp = 2 * H, 2 * W
    MW = Wp
    FLAT = Hp * MW
    MARG = ((MW + 2 + 15) // 16) * 16
    FH = MARG + FLAT + MARG
    rows = 8 if Hp % 8 == 0 else Hp
    STRIP = rows * MW
    NS = Hp // rows
    return Hp, Wp, MW, FLAT, MARG, FH, STRIP, NS


def _bn_fold(gamma, beta, mean, var, conv_bias):
    s = gamma / jnp.sqrt(var + _EPS)
    return s, (conv_bias - mean) * s + beta


def _fused_body(H, W, Cin, C):
    Hp, Wp, _MW, _FLAT, _MARG, _FH, _STRIP, _NS = _dims(H, W)

    offs = [(ky - 1) * _MW + (kx - 1) for ky in range(3) for kx in range(3)]

    def _conv_strip(src0, src1, w_ref, s, m0, m2):
        base = s * _STRIP
        taps = []
        for ky in range(3):
            for kx in range(3):
                off = (ky - 1) * _MW + (kx - 1)
                if off % 2 == 0:
                    t = src0[_MARG + base + off:
                             _MARG + base + off + _STRIP, :]
                else:
                    t = src1[_MARG + 1 + base + off:
                             _MARG + 1 + base + off + _STRIP, :]
                if kx == 0:
                    t = jnp.where(m0, t, jnp.bfloat16(0))
                elif kx == 2:
                    t = jnp.where(m2, t, jnp.bfloat16(0))
                taps.append(t)
        a = jnp.concatenate(taps, axis=1)
        return jnp.dot(a, w_ref[...], preferred_element_type=jnp.float32)

    def _store_stage(dst0, dst1, v):
        dst0[0:_MARG, :] = jnp.zeros((_MARG, C), jnp.bfloat16)
        dst0[_MARG + _FLAT:_FH, :] = jnp.zeros((_MARG, C), jnp.bfloat16)
        dst1[0:_MARG + 8, :] = jnp.zeros((_MARG + 8, C), jnp.bfloat16)
        dst1[_MARG + _FLAT - 8:_FH, :] = jnp.zeros(
            (_FH - (_MARG + _FLAT - 8), C), jnp.bfloat16)
        dst0[_MARG:_MARG + _FLAT, :] = v
        dst1[_MARG + 1:_MARG + 1 + _FLAT, :] = v

    def body(x_ref, wup_ref, bup_ref, w1_ref, b1_ref, w2_ref, b2_ref,
             o_ref, *bufs):
        for i in (0, 1):
            _image(x_ref, wup_ref, bup_ref, w1_ref, b1_ref, w2_ref, b2_ref,
                   o_ref, i, *bufs[4 * i:4 * i + 4])

    def _image(x_ref, wup_ref, bup_ref, w1_ref, b1_ref, w2_ref, b2_ref,
               o_ref, i, f0, f1, g0, g1):
        ci = lax.broadcasted_iota(jnp.int32, (_STRIP, 1), 0) % _MW
        m0 = ci != 0
        m2 = ci != (Wp - 1)

        up = jnp.dot(x_ref[i], wup_ref[...],
                     preferred_element_type=jnp.float32)
        up = jnp.maximum(up + bup_ref[...], 0.0).astype(jnp.bfloat16)
        v = (up.reshape(H, W, 2, 2 * C)
               .transpose(0, 2, 1, 3)
               .reshape(_FLAT, C))
        _store_stage(f0, f1, v)

        g0[0:_MARG, :] = jnp.zeros((_MARG, C), jnp.bfloat16)
        g0[_MARG + _FLAT:_FH, :] = jnp.zeros((_MARG, C), jnp.bfloat16)
        g1[0:_MARG + 8, :] = jnp.zeros((_MARG + 8, C), jnp.bfloat16)
        g1[_MARG + _FLAT - 8:_FH, :] = jnp.zeros(
            (_FH - (_MARG + _FLAT - 8), C), jnp.bfloat16)
        for s in range(_NS):
            acc = _conv_strip(f0, f1, w1_ref, s, m0, m2)
            h = jnp.maximum(acc + b1_ref[...], 0.0).astype(jnp.bfloat16)
            g0[_MARG + s * _STRIP:_MARG + (s + 1) * _STRIP, :] = h
            g1[_MARG + 1 + s * _STRIP:_MARG + 1 + (s + 1) * _STRIP, :] = h

        rows = _STRIP // _MW
        for s in range(_NS):
            acc = _conv_strip(g0, g1, w2_ref, s, m0, m2)
            h = jnp.maximum(acc + b2_ref[...], 0.0)
            o_ref[i, s * _STRIP:(s + 1) * _STRIP, :] = h

    return body


def kernel(x_nhwc, w_up, b_up, g_up, beta_up, m_up, v_up,
           w1, b1, g1, beta1, m1, v1, w2, b2, g2, beta2, m2, v2):
    N, H, W, Cin = x_nhwc.shape
    C = w_up.shape[1]
    Hp, Wp = 2 * H, 2 * W

    s_up, sh_up = _bn_fold(g_up, beta_up, m_up, v_up, b_up)
    wup = (jnp.transpose(w_up, (0, 2, 3, 1)) * s_up).reshape(Cin, 4 * C)
    bup = jnp.tile(sh_up, 4)[None, :]
    s1, bb1 = _bn_fold(g1, beta1, m1, v1, b1)
    s2, bb2 = _bn_fold(g2, beta2, m2, v2, b2)
    w1f = (jnp.transpose(w1, (2, 3, 1, 0)) * s1).reshape(9 * C, C)
    w2f = (jnp.transpose(w2, (2, 3, 1, 0)) * s2).reshape(9 * C, C)

    x2d = x_nhwc.reshape(N, H * W, Cin).astype(jnp.bfloat16)
    wup = wup.astype(jnp.bfloat16)
    w1f = w1f.astype(jnp.bfloat16)
    w2f = w2f.astype(jnp.bfloat16)

    _, _, _, _, _, FH, _, _ = _dims(H, W)

    def full(shape):
        return pl.BlockSpec(shape, lambda n: (0,) * len(shape))

    out = pl.pallas_call(
        _fused_body(H, W, Cin, C),
        out_shape=jax.ShapeDtypeStruct((N, Hp * Wp, C), jnp.float32),
        grid=(N // 2,),
        in_specs=[
            pl.BlockSpec((2, H * W, Cin), lambda n: (n, 0, 0)),
            full((Cin, 4 * C)), full((1, 4 * C)),
            full((9 * C, C)), full((1, C)),
            full((9 * C, C)), full((1, C)),
        ],
        out_specs=pl.BlockSpec((2, Hp * Wp, C), lambda n: (n, 0, 0)),
        scratch_shapes=[pltpu.VMEM((FH, C), jnp.bfloat16) for _ in range(8)],
        compiler_params=pltpu.CompilerParams(
            dimension_semantics=("parallel",),
            vmem_limit_bytes=56 * 1024 * 1024),
    )(x2d, wup, bup, w1f, bb1[None, :], w2f, bb2[None, :])
    return out.reshape(N, Hp, Wp, C)


# speedup vs baseline: 1.0130x; 1.0130x over previous
import jax
import jax.numpy as jnp
from jax import lax
from jax.experimental import pallas as pl
from jax.experimental.pallas import tpu as pltpu

_EPS = 1e-5


def _dims(H, W):
    Hp, Wp = 2 * H, 2 * W
    MW = Wp
    FLAT = Hp * MW
    MARG = ((MW + 2 + 15) // 16) * 16
    FH = MARG + FLAT + MARG
    rows = 8 if Hp % 8 == 0 else Hp
    STRIP = rows * MW
    NS = Hp // rows
    return Hp, Wp, MW, FLAT, MARG, FH, STRIP, NS


def _bn_fold(gamma, beta, mean, var, conv_bias):
    s = gamma / jnp.sqrt(var + _EPS)
    return s, (conv_bias - mean) * s + beta


def _fused_body(H, W, Cin, C):
    Hp, Wp, _MW, _FLAT, _MARG, _FH, _STRIP, _NS = _dims(H, W)

    offs = [(ky - 1) * _MW + (kx - 1) for ky in range(3) for kx in range(3)]

    def _conv_strip(src0, src1, w_ref, s, m0, m2):
        base = s * _STRIP
        taps = []
        for ky in range(3):
            for kx in range(3):
                off = (ky - 1) * _MW + (kx - 1)
                if off % 2 == 0:
                    t = src0[_MARG + base + off:
                             _MARG + base + off + _STRIP, :]
                else:
                    t = src1[_MARG + 1 + base + off:
                             _MARG + 1 + base + off + _STRIP, :]
                if kx == 0:
                    t = jnp.where(m0, t, jnp.bfloat16(0))
                elif kx == 2:
                    t = jnp.where(m2, t, jnp.bfloat16(0))
                taps.append(t)
        a = jnp.concatenate(taps, axis=1)
        return jnp.dot(a, w_ref[...], preferred_element_type=jnp.float32)

    def _store_stage(dst0, dst1, v):
        dst0[0:_MARG, :] = jnp.zeros((_MARG, C), jnp.bfloat16)
        dst0[_MARG + _FLAT:_FH, :] = jnp.zeros((_MARG, C), jnp.bfloat16)
        dst1[0:_MARG + 8, :] = jnp.zeros((_MARG + 8, C), jnp.bfloat16)
        dst1[_MARG + _FLAT - 8:_FH, :] = jnp.zeros(
            (_FH - (_MARG + _FLAT - 8), C), jnp.bfloat16)
        dst0[_MARG:_MARG + _FLAT, :] = v
        dst1[_MARG + 1:_MARG + 1 + _FLAT, :] = v

    def body(x_ref, wup_ref, bup_ref, w1_ref, b1_ref, w2_ref, b2_ref,
             o_ref, *bufs):
        for i in (0, 1):
            _image(x_ref, wup_ref, bup_ref, w1_ref, b1_ref, w2_ref, b2_ref,
                   o_ref, i, *bufs[4 * i:4 * i + 4])

    def _image(x_ref, wup_ref, bup_ref, w1_ref, b1_ref, w2_ref, b2_ref,
               o_ref, i, f0, f1, g0, g1):
        ci = lax.broadcasted_iota(jnp.int32, (_STRIP, 1), 0) % _MW
        m0 = ci != 0
        m2 = ci != (Wp - 1)

        up = jnp.dot(x_ref[i], wup_ref[...],
                     preferred_element_type=jnp.float32)
        up = jnp.maximum(up + bup_ref[...], 0.0).astype(jnp.bfloat16)
        v = (up.reshape(H, W, 2, 2, C)
               .transpose(0, 2, 1, 3, 4)
               .reshape(Hp, Wp, C))
        _store_stage(f0, f1, v.reshape(_FLAT, C))

        g0[0:_MARG, :] = jnp.zeros((_MARG, C), jnp.bfloat16)
        g0[_MARG + _FLAT:_FH, :] = jnp.zeros((_MARG, C), jnp.bfloat16)
        g1[0:_MARG + 8, :] = jnp.zeros((_MARG + 8, C), jnp.bfloat16)
        g1[_MARG + _FLAT - 8:_FH, :] = jnp.zeros(
            (_FH - (_MARG + _FLAT - 8), C), jnp.bfloat16)
        for s in range(_NS):
            acc = _conv_strip(f0, f1, w1_ref, s, m0, m2)
            h = jnp.maximum(acc + b1_ref[...], 0.0).astype(jnp.bfloat16)
            g0[_MARG + s * _STRIP:_MARG + (s + 1) * _STRIP, :] = h
            g1[_MARG + 1 + s * _STRIP:_MARG + 1 + (s + 1) * _STRIP, :] = h

        rows = _STRIP // _MW
        for s in range(_NS):
            acc = _conv_strip(g0, g1, w2_ref, s, m0, m2)
            h = jnp.maximum(acc + b2_ref[...], 0.0)
            o_ref[i, s * _STRIP:(s + 1) * _STRIP, :] = h

    return body


def kernel(x_nhwc, w_up, b_up, g_up, beta_up, m_up, v_up,
           w1, b1, g1, beta1, m1, v1, w2, b2, g2, beta2, m2, v2):
    N, H, W, Cin = x_nhwc.shape
    C = w_up.shape[1]
    Hp, Wp = 2 * H, 2 * W

    s_up, sh_up = _bn_fold(g_up, beta_up, m_up, v_up, b_up)
    wup = (jnp.transpose(w_up, (0, 2, 3, 1)) * s_up).reshape(Cin, 4 * C)
    bup = jnp.tile(sh_up, 4)[None, :]
    s1, bb1 = _bn_fold(g1, beta1, m1, v1, b1)
    s2, bb2 = _bn_fold(g2, beta2, m2, v2, b2)
    w1f = (jnp.transpose(w1, (2, 3, 1, 0)) * s1).reshape(9 * C, C)
    w2f = (jnp.transpose(w2, (2, 3, 1, 0)) * s2).reshape(9 * C, C)

    x2d = x_nhwc.reshape(N, H * W, Cin).astype(jnp.bfloat16)
    wup = wup.astype(jnp.bfloat16)
    w1f = w1f.astype(jnp.bfloat16)
    w2f = w2f.astype(jnp.bfloat16)

    _, _, _, _, _, FH, _, _ = _dims(H, W)

    def full(shape):
        return pl.BlockSpec(shape, lambda n: (0,) * len(shape))

    out = pl.pallas_call(
        _fused_body(H, W, Cin, C),
        out_shape=jax.ShapeDtypeStruct((N, Hp * Wp, C), jnp.float32),
        grid=(N // 2,),
        in_specs=[
            pl.BlockSpec((2, H * W, Cin), lambda n: (n, 0, 0)),
            full((Cin, 4 * C)), full((1, 4 * C)),
            full((9 * C, C)), full((1, C)),
            full((9 * C, C)), full((1, C)),
        ],
        out_specs=pl.BlockSpec((2, Hp * Wp, C), lambda n: (n, 0, 0)),
        scratch_shapes=[pltpu.VMEM((FH, C), jnp.bfloat16) for _ in range(8)],
        compiler_params=pltpu.CompilerParams(
            dimension_semantics=("parallel",),
            vmem_limit_bytes=56 * 1024 * 1024),
    )(x2d, wup, bup, w1f, bb1[None, :], w2f, bb2[None, :])
    return out.reshape(N, Hp, Wp, C)
